# baseline (device time: 360215 ns/iter reference)
import jax
import jax.numpy as jnp
from jax import lax
from jax.experimental import pallas as pl
from jax.experimental.pallas import tpu as pltpu

N_DEV = 8
N_SEG = 8


def kernel(x, w_mat):
    m, k_local = x.shape
    _, n = w_mat.shape
    m_chunk = m // N_DEV
    nh = n // 2
    segw = nh // N_SEG

    def body(x_hbm, w_ref, out_hbm, xbuf, cwb, ccwb, obuf, load_sems,
             store_sems, cw_send, cw_recv, ccw_send, ccw_recv,
             credit_cw, credit_ccw):
        my = lax.axis_index("i")
        left = lax.rem(my - 1 + N_DEV, N_DEV)
        right = lax.rem(my + 1, N_DEV)

        barrier = pltpu.get_barrier_semaphore()
        for nbr in (left, right):
            pl.semaphore_signal(
                barrier, inc=1, device_id=(nbr,),
                device_id_type=pl.DeviceIdType.MESH,
            )
        pl.semaphore_wait(barrier, 2)

        def load(c, slot):
            cp = pltpu.make_async_copy(
                x_hbm.at[pl.ds(c * m_chunk, m_chunk), :], xbuf.at[slot],
                load_sems.at[slot])
            cp.start()
            return cp

        def dot_seg(slot, col0):
            return jnp.dot(
                xbuf[slot], w_ref[:, col0:col0 + segw],
                preferred_element_type=jnp.float32)

        def rc(buf, sl_src, sl_dst, g, send, recv, dst):
            return pltpu.make_async_remote_copy(
                src_ref=buf.at[sl_src, g], dst_ref=buf.at[sl_dst, g],
                send_sem=send.at[sl_src, g], recv_sem=recv.at[sl_dst, g],
                device_id=(dst,), device_id_type=pl.DeviceIdType.MESH,
            )

        l0 = load(lax.rem(my - 1 + N_DEV, N_DEV), 0)
        l1 = load(lax.rem(my + 1, N_DEV), 1)
        r_cw = []
        r_ccw = []
        l0.wait()
        for g in range(N_SEG):
            cwb[0, g] = dot_seg(0, g * segw).astype(jnp.bfloat16)
            r = rc(cwb, 0, 1, g, cw_send, cw_recv, right)
            r.start()
            r_cw.append(r)
        l1.wait()
        for g in range(N_SEG):
            ccwb[0, g] = dot_seg(1, nh + g * segw).astype(jnp.bfloat16)
            r = rc(ccwb, 0, 1, g, ccw_send, ccw_recv, left)
            r.start()
            r_ccw.append(r)

        for t in range(1, N_DEV):
            q = t % 2
            p = 1 - q
            last = t == N_DEV - 1
            l0 = load(lax.rem(my - t - 1 + 2 * N_DEV, N_DEV), 0)
            l1 = load(lax.rem(my + t + 1, N_DEV), 1)
            new_cw = []
            new_ccw = []
            for g in range(N_SEG):
                col_cw = g * segw
                col_ccw = nh + g * segw
                r_cw[g].wait()
                if not last:
                    pl.semaphore_signal(
                        credit_cw, inc=1, device_id=(left,),
                        device_id_type=pl.DeviceIdType.MESH)
                if g == 0:
                    l0.wait()
                if not last:
                    cwb[q, g] = (cwb[q, g].astype(jnp.float32)
                                 + dot_seg(0, col_cw)).astype(jnp.bfloat16)
                    pl.semaphore_wait(credit_cw, 1)
                    r = rc(cwb, q, p, g, cw_send, cw_recv, right)
                    r.start()
                    new_cw.append(r)
                else:
                    acc = cwb[q, g].astype(jnp.float32) + dot_seg(0, col_cw)
                    obuf[0, :, col_cw:col_cw + segw] = (
                        acc * jax.nn.sigmoid(acc))
                r_ccw[g].wait()
                if not last:
                    pl.semaphore_signal(
                        credit_ccw, inc=1, device_id=(right,),
                        device_id_type=pl.DeviceIdType.MESH)
                if g == 0:
                    l1.wait()
                if not last:
                    ccwb[q, g] = (ccwb[q, g].astype(jnp.float32)
                                  + dot_seg(1, col_ccw)).astype(jnp.bfloat16)
                    pl.semaphore_wait(credit_ccw, 1)
                    r = rc(ccwb, q, p, g, ccw_send, ccw_recv, left)
                    r.start()
                    new_ccw.append(r)
                else:
                    acc = (ccwb[q, g].astype(jnp.float32)
                           + dot_seg(1, col_ccw))
                    obuf[1, :, g * segw:(g + 1) * segw] = (
                        acc * jax.nn.sigmoid(acc))
            r_cw = new_cw
            r_ccw = new_ccw

        cp0 = pltpu.make_async_copy(
            obuf.at[0], out_hbm.at[:, pl.ds(0, nh)], store_sems.at[0])
        cp1 = pltpu.make_async_copy(
            obuf.at[1], out_hbm.at[:, pl.ds(nh, nh)], store_sems.at[1])
        cp0.start()
        cp1.start()
        cp0.wait()
        cp1.wait()

    return pl.pallas_call(
        body,
        out_shape=jax.ShapeDtypeStruct((m_chunk, n), jnp.float32),
        in_specs=[
            pl.BlockSpec(memory_space=pl.ANY),
            pl.BlockSpec(memory_space=pltpu.VMEM),
        ],
        out_specs=pl.BlockSpec(memory_space=pl.ANY),
        scratch_shapes=[
            pltpu.VMEM((2, m_chunk, k_local), jnp.float32),
            pltpu.VMEM((2, N_SEG, m_chunk, segw), jnp.bfloat16),
            pltpu.VMEM((2, N_SEG, m_chunk, segw), jnp.bfloat16),
            pltpu.VMEM((2, m_chunk, nh), jnp.float32),
            pltpu.SemaphoreType.DMA((2,)),
            pltpu.SemaphoreType.DMA((2,)),
            pltpu.SemaphoreType.DMA((2, N_SEG)),
            pltpu.SemaphoreType.DMA((2, N_SEG)),
            pltpu.SemaphoreType.DMA((2, N_SEG)),
            pltpu.SemaphoreType.DMA((2, N_SEG)),
            pltpu.SemaphoreType.REGULAR,
            pltpu.SemaphoreType.REGULAR,
        ],
        compiler_params=pltpu.CompilerParams(
            collective_id=0, vmem_limit_bytes=100 * 1024 * 1024),
    )(x, w_mat)


# device time: 351015 ns/iter; 1.0262x vs baseline; 1.0262x over previous
import jax
import jax.numpy as jnp
from jax import lax
from jax.experimental import pallas as pl
from jax.experimental.pallas import tpu as pltpu

N_DEV = 8
N_SEG = 4


def kernel(x, w_mat):
    m, k_local = x.shape
    _, n = w_mat.shape
    m_chunk = m // N_DEV
    nh = n // 2
    segw = nh // N_SEG

    def body(x_hbm, w_ref, out_hbm, xbuf, cwb, ccwb, obuf, load_sems,
             store_sems, cw_send, cw_recv, ccw_send, ccw_recv,
             credit_cw, credit_ccw):
        my = lax.axis_index("i")
        left = lax.rem(my - 1 + N_DEV, N_DEV)
        right = lax.rem(my + 1, N_DEV)

        barrier = pltpu.get_barrier_semaphore()
        for nbr in (left, right):
            pl.semaphore_signal(
                barrier, inc=1, device_id=(nbr,),
                device_id_type=pl.DeviceIdType.MESH,
            )
        pl.semaphore_wait(barrier, 2)

        def load(c, slot):
            cp = pltpu.make_async_copy(
                x_hbm.at[pl.ds(c * m_chunk, m_chunk), :], xbuf.at[slot],
                load_sems.at[slot])
            cp.start()
            return cp

        def dot_seg(slot, col0):
            return jnp.dot(
                xbuf[slot], w_ref[:, col0:col0 + segw],
                preferred_element_type=jnp.float32)

        def rc(buf, sl_src, sl_dst, g, send, recv, dst):
            return pltpu.make_async_remote_copy(
                src_ref=buf.at[sl_src, g], dst_ref=buf.at[sl_dst, g],
                send_sem=send.at[sl_src, g], recv_sem=recv.at[sl_dst, g],
                device_id=(dst,), device_id_type=pl.DeviceIdType.MESH,
            )

        l0 = load(lax.rem(my - 1 + N_DEV, N_DEV), 0)
        l1 = load(lax.rem(my + 1, N_DEV), 1)
        r_cw = []
        r_ccw = []
        l0.wait()
        for g in range(N_SEG):
            cwb[0, g] = dot_seg(0, g * segw).astype(jnp.bfloat16)
            r = rc(cwb, 0, 1, g, cw_send, cw_recv, right)
            r.start()
            r_cw.append(r)
        l1.wait()
        for g in range(N_SEG):
            ccwb[0, g] = dot_seg(1, nh + g * segw).astype(jnp.bfloat16)
            r = rc(ccwb, 0, 1, g, ccw_send, ccw_recv, left)
            r.start()
            r_ccw.append(r)

        for t in range(1, N_DEV):
            q = t % 2
            p = 1 - q
            last = t == N_DEV - 1
            l0 = load(lax.rem(my - t - 1 + 2 * N_DEV, N_DEV), 0)
            l1 = load(lax.rem(my + t + 1, N_DEV), 1)
            new_cw = []
            new_ccw = []
            stores = []
            for g in range(N_SEG):
                col_cw = g * segw
                col_ccw = nh + g * segw
                r_cw[g].wait()
                if not last:
                    pl.semaphore_signal(
                        credit_cw, inc=1, device_id=(left,),
                        device_id_type=pl.DeviceIdType.MESH)
                if g == 0:
                    l0.wait()
                if not last:
                    cwb[q, g] = (cwb[q, g].astype(jnp.float32)
                                 + dot_seg(0, col_cw)).astype(jnp.bfloat16)
                    pl.semaphore_wait(credit_cw, 1)
                    r = rc(cwb, q, p, g, cw_send, cw_recv, right)
                    r.start()
                    new_cw.append(r)
                else:
                    acc = cwb[q, g].astype(jnp.float32) + dot_seg(0, col_cw)
                    obuf[0, :, col_cw:col_cw + segw] = (
                        acc * jax.nn.sigmoid(acc))
                    cp = pltpu.make_async_copy(
                        obuf.at[0, :, pl.ds(col_cw, segw)],
                        out_hbm.at[:, pl.ds(col_cw, segw)],
                        store_sems.at[0, g])
                    cp.start()
                    stores.append(cp)
                r_ccw[g].wait()
                if not last:
                    pl.semaphore_signal(
                        credit_ccw, inc=1, device_id=(right,),
                        device_id_type=pl.DeviceIdType.MESH)
                if g == 0:
                    l1.wait()
                if not last:
                    ccwb[q, g] = (ccwb[q, g].astype(jnp.float32)
                                  + dot_seg(1, col_ccw)).astype(jnp.bfloat16)
                    pl.semaphore_wait(credit_ccw, 1)
                    r = rc(ccwb, q, p, g, ccw_send, ccw_recv, left)
                    r.start()
                    new_ccw.append(r)
                else:
                    acc = (ccwb[q, g].astype(jnp.float32)
                           + dot_seg(1, col_ccw))
                    obuf[1, :, g * segw:(g + 1) * segw] = (
                        acc * jax.nn.sigmoid(acc))
                    cp = pltpu.make_async_copy(
                        obuf.at[1, :, pl.ds(g * segw, segw)],
                        out_hbm.at[:, pl.ds(col_ccw, segw)],
                        store_sems.at[1, g])
                    cp.start()
                    stores.append(cp)
            r_cw = new_cw
            r_ccw = new_ccw

        for cp in stores:
            cp.wait()

    return pl.pallas_call(
        body,
        out_shape=jax.ShapeDtypeStruct((m_chunk, n), jnp.float32),
        in_specs=[
            pl.BlockSpec(memory_space=pl.ANY),
            pl.BlockSpec(memory_space=pltpu.VMEM),
        ],
        out_specs=pl.BlockSpec(memory_space=pl.ANY),
        scratch_shapes=[
            pltpu.VMEM((2, m_chunk, k_local), jnp.float32),
            pltpu.VMEM((2, N_SEG, m_chunk, segw), jnp.bfloat16),
            pltpu.VMEM((2, N_SEG, m_chunk, segw), jnp.bfloat16),
            pltpu.VMEM((2, m_chunk, nh), jnp.float32),
            pltpu.SemaphoreType.DMA((2,)),
            pltpu.SemaphoreType.DMA((2, N_SEG)),
            pltpu.SemaphoreType.DMA((2, N_SEG)),
            pltpu.SemaphoreType.DMA((2, N_SEG)),
            pltpu.SemaphoreType.DMA((2, N_SEG)),
            pltpu.SemaphoreType.DMA((2, N_SEG)),
            pltpu.SemaphoreType.REGULAR,
            pltpu.SemaphoreType.REGULAR,
        ],
        compiler_params=pltpu.CompilerParams(
            collective_id=0, vmem_limit_bytes=100 * 1024 * 1024),
    )(x, w_mat)
